# Initial kernel scaffold; baseline (speedup 1.0000x reference)
import jax
import jax.numpy as jnp
from jax.experimental import pallas as pl


def kernel(x, lW1, lb1, lW2, lb2, lW3, lb3, mW1, mb1, mW2, mb2, mW3, mb3, hW1, hb1, hW2, hb2, hW3, hb3):
    raise NotImplementedError("write your pallas kernel here")


# speedup vs baseline: 1.5552x; 1.5552x over previous
import functools

import jax
import jax.numpy as jnp
from jax.experimental import pallas as pl
from jax.experimental.pallas import tpu as pltpu

N_ROWS = 8192
IN_DIM = 769
H = 64
A = 32
S = 64
OUT_W = A * S

TILE = 256
GRID = N_ROWS // TILE


def _fused_body(x_ref, w1_ref, b1_ref, w2_ref, b2_ref, w3_ref, m4_ref, o_ref):
    xt = x_ref[...]
    g = xt[:, IN_DIM - 1:IN_DIM]
    ml = ((g >= 0.0) & (g < 0.5)).astype(jnp.float32)
    mm = ((g >= 0.5) & (g < 0.75)).astype(jnp.float32)
    mh = ((g >= 0.75) & (g <= 1.0)).astype(jnp.float32)

    h1 = jnp.dot(xt.astype(jnp.bfloat16), w1_ref[...],
                 preferred_element_type=jnp.float32) + b1_ref[...]
    h1 = jnp.maximum(h1, 0.0)
    h1m = jnp.concatenate(
        [h1[:, 0:H] * ml, h1[:, H:2 * H] * mm, h1[:, 2 * H:3 * H] * mh], axis=1)

    h2 = jnp.dot(h1m.astype(jnp.bfloat16), w2_ref[...],
                 preferred_element_type=jnp.float32) + b2_ref[...]
    h2 = jnp.maximum(h2, 0.0)

    zeros_pad = jnp.zeros((TILE, 256 - 3 * H - 3), jnp.float32)
    aug = jnp.concatenate(
        [h2[:, 0:H] * ml, h2[:, H:2 * H] * mm, h2[:, 2 * H:3 * H] * mh,
         ml, mm, mh, zeros_pad], axis=1)

    y = jnp.dot(aug.astype(jnp.bfloat16), w3_ref[...],
                preferred_element_type=jnp.float32)
    y = jnp.maximum(y, 0.0)

    y8 = y.reshape(TILE * 8, 256)
    o_ref[...] = jnp.dot(y8.astype(jnp.bfloat16), m4_ref[...],
                         preferred_element_type=jnp.float32)


@functools.partial(jax.jit, static_argnames=())
def _prep_and_run(x, lW1, lb1, lW2, lb2, lW3, lb3,
                  mW1, mb1, mW2, mb2, mW3, mb3,
                  hW1, hb1, hW2, hb2, hW3, hb3):
    f32 = jnp.float32
    bf16 = jnp.bfloat16

    w1c = jnp.concatenate([lW1, mW1, hW1], axis=1).astype(bf16)
    b1c = jnp.concatenate([lb1, mb1, hb1]).reshape(1, 3 * H).astype(f32)

    zb = jnp.zeros((H, H), f32)
    w2bd = jnp.block([[lW2, zb, zb], [zb, mW2, zb], [zb, zb, hW2]]).astype(bf16)
    b2c = jnp.concatenate([lb2, mb2, hb2]).reshape(1, 3 * H).astype(f32)

    w3v = jnp.concatenate(
        [lW3, mW3, hW3, lb3.reshape(1, OUT_W), mb3.reshape(1, OUT_W),
         hb3.reshape(1, OUT_W), jnp.zeros((256 - 3 * H - 3, OUT_W), f32)],
        axis=0).astype(bf16)

    jj = jax.lax.broadcasted_iota(jnp.int32, (256, 256), 0)
    kk = jax.lax.broadcasted_iota(jnp.int32, (256, 256), 1)
    m4 = (((jj // S) == (kk // S)) & ((jj % S) + (kk % S) <= S - 1)).astype(bf16)

    out = pl.pallas_call(
        _fused_body,
        grid=(GRID,),
        in_specs=[
            pl.BlockSpec((TILE, IN_DIM), lambda t: (t, 0)),
            pl.BlockSpec((IN_DIM, 3 * H), lambda t: (0, 0)),
            pl.BlockSpec((1, 3 * H), lambda t: (0, 0)),
            pl.BlockSpec((3 * H, 3 * H), lambda t: (0, 0)),
            pl.BlockSpec((1, 3 * H), lambda t: (0, 0)),
            pl.BlockSpec((256, OUT_W), lambda t: (0, 0)),
            pl.BlockSpec((256, 256), lambda t: (0, 0)),
        ],
        out_specs=pl.BlockSpec((TILE * 8, 256), lambda t: (t, 0)),
        out_shape=jax.ShapeDtypeStruct((N_ROWS * 8, 256), f32),
    )(x, w1c, b1c, w2bd, b2c, w3v, m4)
    return out.reshape(N_ROWS, A, S)


def kernel(x, lW1, lb1, lW2, lb2, lW3, lb3, mW1, mb1, mW2, mb2, mW3, mb3,
           hW1, hb1, hW2, hb2, hW3, hb3):
    return _prep_and_run(x, lW1, lb1, lW2, lb2, lW3, lb3,
                         mW1, mb1, mW2, mb2, mW3, mb3,
                         hW1, hb1, hW2, hb2, hW3, hb3)
